# Initial kernel scaffold; baseline (speedup 1.0000x reference)
import jax
import jax.numpy as jnp
from jax.experimental import pallas as pl


def kernel(indices, embed, w0, b0, w1, b1, w2, b2):
    raise NotImplementedError("write your pallas kernel here")


# speedup vs baseline: 7.8137x; 7.8137x over previous
import functools

import jax
import jax.numpy as jnp
from jax import lax
from jax.experimental import pallas as pl
from jax.experimental.pallas import tpu as pltpu
from jax.experimental.pallas import tpu_sc as plsc

B = 16384
F = 26
V = 100000
D = 16
FD = F * D
H0, H1 = 400, 400

NC, NS = 2, 16
NW = NC * NS
BF = B * F
RPW = BF // NW
CW = 128
CH = RPW // CW
GRP = 4
NGROUPS = CH // GRP
GROWS = GRP * CW


def _sc_gather_body(table_hbm, idx_hbm, out_hbm, idx_v, buf0, buf1,
                    gsem0, gsem1):
    wid = lax.axis_index("s") * NC + lax.axis_index("c")
    base = wid * RPW
    pltpu.sync_copy(idx_hbm.at[wid], idx_v)

    bufs = (buf0, buf1)
    gsems = (gsem0, gsem1)

    def fire_group(gi, s):
        for c in range(GRP):
            j = gi * GRP + c
            pltpu.async_copy(table_hbm.at[idx_v.at[j]],
                             bufs[s].at[pl.ds(c * CW, CW)], gsems[s])

    fire_group(0, 0)

    def outer(k, _):
        for s in range(2):
            gi = 2 * k + s

            @pl.when(gi + 1 < NGROUPS)
            def _():
                fire_group(gi + 1, (s + 1) % 2)

            for c in range(GRP):
                j = gi * GRP + c
                pltpu.make_async_copy(table_hbm.at[idx_v.at[j]],
                                      bufs[s].at[pl.ds(c * CW, CW)],
                                      gsems[s]).wait()
            pltpu.sync_copy(bufs[s],
                            out_hbm.at[pl.ds(base + gi * GROWS, GROWS)])
        return _

    lax.fori_loop(0, NGROUPS // 2, outer, None)


def _sc_gather(table, idx3d):
    mesh = plsc.VectorSubcoreMesh(core_axis_name="c", subcore_axis_name="s")
    k = functools.partial(
        pl.kernel, mesh=mesh,
        out_type=jax.ShapeDtypeStruct((BF, D), jnp.float32),
        compiler_params=pltpu.CompilerParams(use_tc_tiling_on_sc=False),
        scratch_types=[
            pltpu.VMEM((CH, CW), jnp.int32),
            pltpu.VMEM((GROWS, D), jnp.float32),
            pltpu.VMEM((GROWS, D), jnp.float32),
            pltpu.SemaphoreType.DMA,
            pltpu.SemaphoreType.DMA,
        ],
    )(_sc_gather_body)
    return k(table, idx3d)


def _mlp_body(x_ref, w0_ref, b0_ref, w1_ref, b1_ref, w2_ref, b2_ref, o_ref):
    x = x_ref[...]
    h = jnp.dot(x, w0_ref[...], preferred_element_type=jnp.float32)
    h = jnp.maximum(h + b0_ref[...], 0.0)
    h = jnp.dot(h, w1_ref[...], preferred_element_type=jnp.float32)
    h = jnp.maximum(h + b1_ref[...], 0.0)
    l = jnp.dot(h, w2_ref[...], preferred_element_type=jnp.float32)
    l = l + b2_ref[...]
    o_ref[...] = jax.nn.sigmoid(l)


MB = 2048


def _mlp(xw, w0, b0, w1, b1, w2, b2):
    return pl.pallas_call(
        _mlp_body,
        grid=(B // MB,),
        in_specs=[
            pl.BlockSpec((MB, FD), lambda i: (i, 0)),
            pl.BlockSpec((FD, H0), lambda i: (0, 0)),
            pl.BlockSpec((1, H0), lambda i: (0, 0)),
            pl.BlockSpec((H0, H1), lambda i: (0, 0)),
            pl.BlockSpec((1, H1), lambda i: (0, 0)),
            pl.BlockSpec((H1, 1), lambda i: (0, 0)),
            pl.BlockSpec((1, 1), lambda i: (0, 0)),
        ],
        out_specs=pl.BlockSpec((MB, 1), lambda i: (i, 0)),
        out_shape=jax.ShapeDtypeStruct((B, 1), jnp.float32),
    )(xw, w0, b0.reshape(1, H0), w1, b1.reshape(1, H1), w2,
      b2.reshape(1, 1))


def kernel(indices, embed, w0, b0, w1, b1, w2, b2):
    flat_idx = indices.astype(jnp.int32) + (
        jnp.arange(F, dtype=jnp.int32) * V)[None, :]
    idx3d = flat_idx.reshape(NW, CH, CW)
    table = embed.reshape(F * V, D)
    gathered = _sc_gather(table, idx3d)
    xw = gathered.reshape(B, FD)
    out = _mlp(xw, w0, b0, w1, b1, w2, b2)
    return out[:, 0]
